# baseline (speedup 1.0000x reference)
import functools

import jax
import jax.numpy as jnp
from jax import lax
from jax.experimental import pallas as pl
from jax.experimental.pallas import tpu as pltpu
from jax.experimental.pallas import tpu_sc as plsc

_VOCAB = 30528
_D = 768
_S = 2048
_B = 4
_EPS = 1e-12

_NC = 2
_NS = 16
_NW = _NC * _NS
_NTOK = _B * _S
_TPW = _NTOK // _NW
_C = 32
_NCHUNK = _TPW // _C
_NV = _D // 16


def _rsqrt_vec(v):
    bits = lax.bitcast_convert_type(v, jnp.int32)
    y = lax.bitcast_convert_type(jnp.int32(0x5F3759DF) - (bits >> 1), jnp.float32)
    half = v * 0.5
    for _ in range(3):
        y = y * (1.5 - half * y * y)
    return y


def _sc_body(ids_hbm, tids_hbm, word_hbm, pos_hbm, type_hbm, gamma_hbm,
             beta_hbm, out_hbm, idx_v, tid_v, wbuf, pbuf, tbuf, gbuf, bbuf,
             sem):
    cid = lax.axis_index("c")
    sid = lax.axis_index("s")
    wid = sid * _NC + cid
    tok0 = wid * _TPW
    s0 = lax.rem(tok0, _S)

    pltpu.sync_copy(gamma_hbm, gbuf)
    pltpu.sync_copy(beta_hbm, bbuf)

    def chunk_body(k):
        base = tok0 + k * _C
        sb = s0 + k * _C
        pltpu.sync_copy(ids_hbm.at[pl.ds(base, _C)], idx_v)
        pltpu.sync_copy(tids_hbm.at[pl.ds(base, _C)], tid_v)
        pltpu.async_copy(word_hbm.at[idx_v], wbuf, sem).wait()
        pltpu.async_copy(type_hbm.at[tid_v], tbuf, sem).wait()
        pltpu.sync_copy(pos_hbm.at[pl.ds(sb, _C)], pbuf)

        def token_body(i):
            acc = jnp.zeros((16,), jnp.float32)
            acc2 = jnp.zeros((16,), jnp.float32)
            for j in range(_NV):
                sl = pl.ds(16 * j, 16)
                x = wbuf[i, sl] + (pbuf[i, sl] + tbuf[i, sl])
                wbuf[i, sl] = x
                acc = acc + x
                acc2 = acc2 + x * x
            tot = jnp.sum(acc)
            tot2 = jnp.sum(acc2)
            mean = tot * (1.0 / _D)
            var = tot2 * (1.0 / _D) - mean * mean
            inv = _rsqrt_vec(jnp.full((16,), var + _EPS, jnp.float32))
            meanb = jnp.full((16,), mean, jnp.float32)
            for j in range(_NV):
                sl = pl.ds(16 * j, 16)
                x = wbuf[i, sl]
                wbuf[i, sl] = (x - meanb) * inv * gbuf[sl] + bbuf[sl]

        lax.fori_loop(0, _C, lambda i, _: (token_body(i), 0)[1], 0)
        pltpu.sync_copy(wbuf, out_hbm.at[pl.ds(base, _C)])

    lax.fori_loop(0, _NCHUNK, lambda k, _: (chunk_body(k), 0)[1], 0)


@jax.jit
def _run(ids, tids, word_emb, pos_emb, type_emb, gamma, beta):
    mesh = plsc.VectorSubcoreMesh(
        core_axis_name="c", subcore_axis_name="s", num_cores=_NC,
        num_subcores=_NS)
    f = pl.kernel(
        _sc_body,
        out_type=jax.ShapeDtypeStruct((_NTOK, _D), jnp.float32),
        mesh=mesh,
        compiler_params=pltpu.CompilerParams(needs_layout_passes=False),
        scratch_types=[
            pltpu.VMEM((_C,), jnp.int32),
            pltpu.VMEM((_C,), jnp.int32),
            pltpu.VMEM((_C, _D), jnp.float32),
            pltpu.VMEM((_C, _D), jnp.float32),
            pltpu.VMEM((_C, _D), jnp.float32),
            pltpu.VMEM((_D,), jnp.float32),
            pltpu.VMEM((_D,), jnp.float32),
            pltpu.SemaphoreType.DMA,
        ],
    )
    return f(ids, tids, word_emb, pos_emb, type_emb, gamma, beta)


def kernel(input_ids, token_type_ids, word_emb, pos_emb, type_emb, gamma,
           beta):
    ids = input_ids.reshape(_NTOK).astype(jnp.int32)
    tids = token_type_ids.reshape(_NTOK).astype(jnp.int32)
    out = _run(ids, tids, word_emb, pos_emb, type_emb, gamma, beta)
    return out.reshape(_B, _S, _D)


# speedup vs baseline: 1.1365x; 1.1365x over previous
import jax
import jax.numpy as jnp
from jax import lax
from jax.experimental import pallas as pl
from jax.experimental.pallas import tpu as pltpu
from jax.experimental.pallas import tpu_sc as plsc

_VOCAB = 30528
_D = 768
_S = 2048
_B = 4
_EPS = 1e-12

_NC = 2
_NS = 16
_NW = _NC * _NS
_NTOK = _B * _S
_TPW = _NTOK // _NW
_C = 16
_NCHUNK = _TPW // _C
_NV = _D // 16


def _rsqrt_vec(v):
    bits = lax.bitcast_convert_type(v, jnp.int32)
    y = lax.bitcast_convert_type(jnp.int32(0x5F3759DF) - (bits >> 1), jnp.float32)
    half = v * 0.5
    for _ in range(2):
        y = y * (1.5 - half * y * y)
    return y


def _sc_body(ids_hbm, tids_hbm, word_hbm, pos_hbm, type_hbm, gamma_hbm,
             beta_hbm, out_hbm,
             idx0, idx1, tid0, tid1,
             wbuf0, wbuf1, wbuf2, pbuf0, pbuf1, tbuf0, tbuf1,
             gbuf, bbuf, mbuf, ibuf,
             wsem0, wsem1, wsem2, tsem0, tsem1, psem0, psem1,
             osem0, osem1, osem2):
    idx = [idx0, idx1]
    tid = [tid0, tid1]
    wbuf = [wbuf0, wbuf1, wbuf2]
    pbuf = [pbuf0, pbuf1]
    tbuf = [tbuf0, tbuf1]
    wsem = [wsem0, wsem1, wsem2]
    tsem = [tsem0, tsem1]
    psem = [psem0, psem1]
    osem = [osem0, osem1, osem2]

    cid = lax.axis_index("c")
    sid = lax.axis_index("s")
    wid = sid * _NC + cid
    tok0 = wid * _TPW
    s0 = lax.rem(tok0, _S)

    pltpu.sync_copy(gamma_hbm, gbuf)
    pltpu.sync_copy(beta_hbm, bbuf)

    gdesc = [None, None, None]
    odesc = [None, None, None]

    def fire(kk):
        b2 = kk % 2
        b3 = kk % 3
        base = tok0 + kk * _C
        sb = s0 + kk * _C
        pltpu.sync_copy(ids_hbm.at[pl.ds(base, _C)], idx[b2])
        pltpu.sync_copy(tids_hbm.at[pl.ds(base, _C)], tid[b2])
        if odesc[b3] is not None:
            odesc[b3].wait()
            odesc[b3] = None
        dw = pltpu.async_copy(word_hbm.at[idx[b2]], wbuf[b3], wsem[b3])
        dt = pltpu.async_copy(type_hbm.at[tid[b2]], tbuf[b2], tsem[b2])
        dp = pltpu.async_copy(pos_hbm.at[pl.ds(sb, _C)], pbuf[b2], psem[b2])
        gdesc[b3] = (dw, dt, dp)

    fire(0)
    fire(1)

    for kk in range(_NCHUNK):
        b2 = kk % 2
        b3 = kk % 3
        base = tok0 + kk * _C
        for d in gdesc[b3]:
            d.wait()
        gdesc[b3] = None

        wb = wbuf[b3]
        pb = pbuf[b2]
        tb = tbuf[b2]

        def _pass_a(i):
            accs = [jnp.zeros((16,), jnp.float32) for _ in range(4)]
            acc2s = [jnp.zeros((16,), jnp.float32) for _ in range(4)]
            for j in range(_NV):
                sl = pl.ds(16 * j, 16)
                x = wb[i, sl] + (pb[i, sl] + tb[i, sl])
                wb[i, sl] = x
                accs[j % 4] = accs[j % 4] + x
                acc2s[j % 4] = acc2s[j % 4] + x * x
            acc = (accs[0] + accs[1]) + (accs[2] + accs[3])
            acc2 = (acc2s[0] + acc2s[1]) + (acc2s[2] + acc2s[3])
            tot = jnp.sum(acc)
            tot2 = jnp.sum(acc2)
            mean = tot * (1.0 / _D)
            var = tot2 * (1.0 / _D) - mean * mean
            inv = _rsqrt_vec(jnp.full((16,), var + _EPS, jnp.float32))
            mbuf[i, :] = jnp.full((16,), mean, jnp.float32)
            ibuf[i, :] = inv

        lax.fori_loop(0, _C, lambda i, _: (_pass_a(i), 0)[1], 0)

        for h in range(0, _C, 8):
            mb = [mbuf[i, :] for i in range(h, h + 8)]
            iv = [ibuf[i, :] for i in range(h, h + 8)]

            @plsc.parallel_loop(0, _NV)
            def _pass_b(j):
                sl = pl.ds(j * 16, 16)
                g = gbuf[sl]
                bt = bbuf[sl]
                for i in range(8):
                    wb[h + i, sl] = (wb[h + i, sl] - mb[i]) * iv[i] * g + bt

        odesc[b3] = pltpu.async_copy(wb, out_hbm.at[pl.ds(base, _C)],
                                     osem[b3])
        if kk + 2 < _NCHUNK:
            fire(kk + 2)

    for d in odesc:
        if d is not None:
            d.wait()


@jax.jit
def _run(ids, tids, word_emb, pos_emb, type_emb, gamma, beta):
    mesh = plsc.VectorSubcoreMesh(
        core_axis_name="c", subcore_axis_name="s", num_cores=_NC,
        num_subcores=_NS)
    f = pl.kernel(
        _sc_body,
        out_type=jax.ShapeDtypeStruct((_NTOK, _D), jnp.float32),
        mesh=mesh,
        compiler_params=pltpu.CompilerParams(needs_layout_passes=False),
        scratch_types=[
            pltpu.VMEM((_C,), jnp.int32),
            pltpu.VMEM((_C,), jnp.int32),
            pltpu.VMEM((_C,), jnp.int32),
            pltpu.VMEM((_C,), jnp.int32),
            pltpu.VMEM((_C, _D), jnp.float32),
            pltpu.VMEM((_C, _D), jnp.float32),
            pltpu.VMEM((_C, _D), jnp.float32),
            pltpu.VMEM((_C, _D), jnp.float32),
            pltpu.VMEM((_C, _D), jnp.float32),
            pltpu.VMEM((_C, _D), jnp.float32),
            pltpu.VMEM((_C, _D), jnp.float32),
            pltpu.VMEM((_D,), jnp.float32),
            pltpu.VMEM((_D,), jnp.float32),
            pltpu.VMEM((_C, 16), jnp.float32),
            pltpu.VMEM((_C, 16), jnp.float32),
        ] + [pltpu.SemaphoreType.DMA] * 10,
    )
    return f(ids, tids, word_emb, pos_emb, type_emb, gamma, beta)


def kernel(input_ids, token_type_ids, word_emb, pos_emb, type_emb, gamma,
           beta):
    ids = input_ids.reshape(_NTOK).astype(jnp.int32)
    tids = token_type_ids.reshape(_NTOK).astype(jnp.int32)
    out = _run(ids, tids, word_emb, pos_emb, type_emb, gamma, beta)
    return out.reshape(_B, _S, _D)
